# baseline (speedup 1.0000x reference)
import functools

import jax
import jax.numpy as jnp
from jax import lax
from jax.experimental import pallas as pl
from jax.experimental.pallas import tpu as pltpu
from jax.experimental.pallas import tpu_sc as plsc

N = 10000
E = 320000
D = 128
K = 10
ALPHA = 0.1

NC = 1
NS = 16
NW = NC * NS
NPAD = 10240
RPT = NPAD // NS
EPT = E // NW + 480
B = 128
NB = EPT // B
BB = 64
NBB = EPT // BB
PERROW = B // BB
NROW = 4
NIDX = 8
GLAG = 2
SLAG = 3
IDXA = 5
NG = NBB // NIDX

_mesh = plsc.VectorSubcoreMesh(core_axis_name="c", subcore_axis_name="s",
                               num_cores=NC)


@functools.partial(
    pl.kernel,
    out_type=jax.ShapeDtypeStruct((NPAD, D), jnp.float32),
    mesh=_mesh,
    compiler_params=pltpu.CompilerParams(use_tc_tiling_on_sc=False),
    scratch_types=[
        pltpu.VMEM((NIDX, BB), jnp.int32),
        pltpu.VMEM((NIDX, BB), jnp.int32),
        pltpu.VMEM((NROW, BB, D), jnp.float32),
        pltpu.VMEM_SHARED((NPAD, D), jnp.float32),
        pltpu.SemaphoreType.DMA((NIDX,)),
        pltpu.SemaphoreType.DMA((NROW,)),
        pltpu.SemaphoreType.DMA((NROW,)),
    ],
)
def _scatter(g_hbm, src3, dst3, zrows, out,
             sbuf, dbuf, rows, agg_sh, sem_i, sem_g, sem_s):
    s = lax.axis_index("s")
    wid = s

    def idx_start(bb, h, i):
        pltpu.async_copy(src3.at[wid, bb, pl.ds(h * BB, BB)], sbuf.at[i],
                         sem_i.at[i])
        pltpu.async_copy(dst3.at[wid, bb, pl.ds(h * BB, BB)], dbuf.at[i],
                         sem_i.at[i])

    def idx_wait(bb, h, i):
        pltpu.make_async_copy(src3.at[wid, bb, pl.ds(h * BB, BB)],
                              sbuf.at[i], sem_i.at[i]).wait()
        pltpu.make_async_copy(dst3.at[wid, bb, pl.ds(h * BB, BB)],
                              dbuf.at[i], sem_i.at[i]).wait()

    def g_start(j, i):
        pltpu.async_copy(g_hbm.at[sbuf.at[i]], rows.at[j], sem_g.at[j])

    def g_wait(j, i):
        pltpu.make_async_copy(g_hbm.at[sbuf.at[i]], rows.at[j],
                              sem_g.at[j]).wait()

    def s_start(j, i):
        pltpu.async_copy(rows.at[j], agg_sh.at[dbuf.at[i]], sem_s.at[j],
                         add=True)

    def s_wait(j, i):
        pltpu.make_async_copy(rows.at[j], agg_sh.at[dbuf.at[i]],
                              sem_s.at[j]).wait()

    def slots(b):
        return b % NROW, b % NIDX

    def step(base_bb, t, b_static):
        if b_static is None or b_static - GLAG >= 0:
            g_wait(*slots(t - GLAG))
            s_start(*slots(t - GLAG))
        if b_static is None or b_static - SLAG >= 0:
            s_wait(*slots(t - SLAG))
        if b_static is None or b_static + 1 < NBB:
            idx_wait(base_bb + (t + 1) // PERROW, (t + 1) % PERROW,
                     (t + 1) % NIDX)
            g_start(*slots(t + 1))
        if b_static is None or b_static + IDXA < NBB:
            idx_start(base_bb + (t + IDXA) // PERROW, (t + IDXA) % PERROW,
                      (t + IDXA) % NIDX)

    for b in range(IDXA):
        idx_start(b // PERROW, b % PERROW, b)
    pltpu.sync_copy(zrows, agg_sh.at[pl.ds(s * RPT, RPT)])
    plsc.subcore_barrier()
    idx_wait(0, 0, 0)
    g_start(*slots(0))
    for t in range(NIDX):
        step(0, t, t)

    def _grp(grp):
        for t in range(NIDX):
            step(grp * (NIDX // PERROW), t, None)

    pl.loop(1, NG - 1)(_grp)
    for t in range(NIDX):
        step((NG - 1) * (NIDX // PERROW), t, NBB - NIDX + t)
    for b in range(NBB - GLAG, NBB):
        g_wait(*slots(b))
        s_start(*slots(b))
    for b in range(NBB - SLAG, NBB):
        s_wait(*slots(b))
    plsc.subcore_barrier()
    pltpu.sync_copy(agg_sh.at[pl.ds(s * RPT, RPT)],
                    out.at[pl.ds(s * RPT, RPT)])


def _mlp_body(x_ref, w1_ref, b1_ref, w2_ref, b2_ref, o_ref):
    h = lax.dot_general(x_ref[...], w1_ref[...], (((1,), (1,)), ((), ())),
                        preferred_element_type=jnp.float32)
    h = jnp.maximum(h + b1_ref[...], 0.0)
    o_ref[...] = lax.dot_general(h, w2_ref[...], (((1,), (1,)), ((), ())),
                                 preferred_element_type=jnp.float32) + b2_ref[...]


_mlp = pl.pallas_call(
    _mlp_body, out_shape=jax.ShapeDtypeStruct((N, D), jnp.float32))


def _prep_body(degpair_ref, h0_ref, dinv_ref, g0_ref):
    deg = degpair_ref[0:N, 0:1] + 1.0
    dinv = lax.rsqrt(deg)
    dinv_ref[...] = dinv
    g0_ref[...] = dinv * h0_ref[...]


_prep = pl.pallas_call(
    _prep_body,
    out_shape=(jax.ShapeDtypeStruct((N, 1), jnp.float32),
               jax.ShapeDtypeStruct((N, D), jnp.float32)))


def _update_body(aggpair_ref, gprev_ref, h0_ref, dinv_ref, h_ref, g_ref):
    ssum = aggpair_ref[0:N, :] + gprev_ref[...]
    h = (1.0 - ALPHA) * (dinv_ref[...] * ssum) + ALPHA * h0_ref[...]
    h_ref[...] = h
    g_ref[...] = dinv_ref[...] * h


_update = pl.pallas_call(
    _update_body,
    out_shape=(jax.ShapeDtypeStruct((N, D), jnp.float32),
               jax.ShapeDtypeStruct((N, D), jnp.float32)))


def kernel(x, edge_index, W1, b1, W2, b2):
    src = edge_index[0].astype(jnp.int32)
    dst = edge_index[1].astype(jnp.int32)
    pad = EPT - E // NW
    src3 = jnp.pad(src.reshape(NW, E // NW), ((0, 0), (0, pad)),
                   constant_values=0).reshape(NW, NB, B)
    dst3 = jnp.pad(dst.reshape(NW, E // NW), ((0, 0), (0, pad)),
                   constant_values=NPAD - 1).reshape(NW, NB, B)
    zrows = jnp.zeros((RPT, D), jnp.float32)
    ones_tbl = jnp.ones((N, D), jnp.float32)

    h0 = _mlp(x, W1, b1[None, :], W2, b2[None, :])

    def body(k, carry):
        h, g, dinv = carry

        agg = _scatter(g, src3, dst3, zrows)

        def first(_):
            dinv0, g0 = _prep(agg, h0)
            return (h0, g0, dinv0)

        def later(_):
            h2, g2 = _update(agg, g, h0, dinv)
            return (h2, g2, dinv)

        return lax.cond(k == 0, first, later, None)

    init = (h0, ones_tbl, jnp.zeros((N, 1), jnp.float32))
    h, _, _ = lax.fori_loop(0, K + 1, body, init)
    return h


# speedup vs baseline: 1.0186x; 1.0186x over previous
import functools

import jax
import jax.numpy as jnp
from jax import lax
from jax.experimental import pallas as pl
from jax.experimental.pallas import tpu as pltpu
from jax.experimental.pallas import tpu_sc as plsc

N = 10000
E = 320000
D = 128
K = 10
ALPHA = 0.1

NC = 1
NS = 16
NW = NC * NS
NPAD = 10240
RPT = NPAD // NS
EPT = E // NW + 480
B = 128
NB = EPT // B
BB = 64
NBB = EPT // BB
PERROW = B // BB
NROW = 5
NIDX = 10
GLAG = 3
SLAG = 4
IDXA = 6
NG = NBB // NIDX

_mesh = plsc.VectorSubcoreMesh(core_axis_name="c", subcore_axis_name="s",
                               num_cores=NC)


@functools.partial(
    pl.kernel,
    out_type=jax.ShapeDtypeStruct((NPAD, D), jnp.float32),
    mesh=_mesh,
    compiler_params=pltpu.CompilerParams(use_tc_tiling_on_sc=False),
    scratch_types=[
        pltpu.VMEM((NIDX, BB), jnp.int32),
        pltpu.VMEM((NIDX, BB), jnp.int32),
        pltpu.VMEM((NROW, BB, D), jnp.float32),
        pltpu.VMEM_SHARED((NPAD, D), jnp.float32),
        pltpu.SemaphoreType.DMA((NIDX,)),
        pltpu.SemaphoreType.DMA((NROW,)),
        pltpu.SemaphoreType.DMA((NROW,)),
    ],
)
def _scatter(g_hbm, src3, dst3, zrows, out,
             sbuf, dbuf, rows, agg_sh, sem_i, sem_g, sem_s):
    s = lax.axis_index("s")
    wid = s

    def idx_start(bb, h, i):
        pltpu.async_copy(src3.at[wid, bb, pl.ds(h * BB, BB)], sbuf.at[i],
                         sem_i.at[i])
        pltpu.async_copy(dst3.at[wid, bb, pl.ds(h * BB, BB)], dbuf.at[i],
                         sem_i.at[i])

    def idx_wait(bb, h, i):
        pltpu.make_async_copy(src3.at[wid, bb, pl.ds(h * BB, BB)],
                              sbuf.at[i], sem_i.at[i]).wait()
        pltpu.make_async_copy(dst3.at[wid, bb, pl.ds(h * BB, BB)],
                              dbuf.at[i], sem_i.at[i]).wait()

    def g_start(j, i):
        pltpu.async_copy(g_hbm.at[sbuf.at[i]], rows.at[j], sem_g.at[j])

    def g_wait(j, i):
        pltpu.make_async_copy(g_hbm.at[sbuf.at[i]], rows.at[j],
                              sem_g.at[j]).wait()

    def s_start(j, i):
        pltpu.async_copy(rows.at[j], agg_sh.at[dbuf.at[i]], sem_s.at[j],
                         add=True)

    def s_wait(j, i):
        pltpu.make_async_copy(rows.at[j], agg_sh.at[dbuf.at[i]],
                              sem_s.at[j]).wait()

    def slots(b):
        return b % NROW, b % NIDX

    def step(base_bb, t, b_static):
        if b_static is None or b_static - GLAG >= 0:
            g_wait(*slots(t - GLAG))
            s_start(*slots(t - GLAG))
        if b_static is None or b_static - SLAG >= 0:
            s_wait(*slots(t - SLAG))
        if b_static is None or b_static + 1 < NBB:
            idx_wait(base_bb + (t + 1) // PERROW, (t + 1) % PERROW,
                     (t + 1) % NIDX)
            g_start(*slots(t + 1))
        if b_static is None or b_static + IDXA < NBB:
            idx_start(base_bb + (t + IDXA) // PERROW, (t + IDXA) % PERROW,
                      (t + IDXA) % NIDX)

    for b in range(IDXA):
        idx_start(b // PERROW, b % PERROW, b)
    pltpu.sync_copy(zrows, agg_sh.at[pl.ds(s * RPT, RPT)])
    plsc.subcore_barrier()
    idx_wait(0, 0, 0)
    g_start(*slots(0))
    for t in range(NIDX):
        step(0, t, t)

    def _grp(grp):
        for t in range(NIDX):
            step(grp * (NIDX // PERROW), t, None)

    pl.loop(1, NG - 1)(_grp)
    for t in range(NIDX):
        step((NG - 1) * (NIDX // PERROW), t, NBB - NIDX + t)
    for b in range(NBB - GLAG, NBB):
        g_wait(*slots(b))
        s_start(*slots(b))
    for b in range(NBB - SLAG, NBB):
        s_wait(*slots(b))
    plsc.subcore_barrier()
    pltpu.sync_copy(agg_sh.at[pl.ds(s * RPT, RPT)],
                    out.at[pl.ds(s * RPT, RPT)])


def _mlp_body(x_ref, w1_ref, b1_ref, w2_ref, b2_ref, o_ref):
    h = lax.dot_general(x_ref[...], w1_ref[...], (((1,), (1,)), ((), ())),
                        preferred_element_type=jnp.float32)
    h = jnp.maximum(h + b1_ref[...], 0.0)
    o_ref[...] = lax.dot_general(h, w2_ref[...], (((1,), (1,)), ((), ())),
                                 preferred_element_type=jnp.float32) + b2_ref[...]


_mlp = pl.pallas_call(
    _mlp_body, out_shape=jax.ShapeDtypeStruct((N, D), jnp.float32))


def _prep_body(degpair_ref, h0_ref, dinv_ref, g0_ref):
    deg = degpair_ref[0:N, 0:1] + 1.0
    dinv = lax.rsqrt(deg)
    dinv_ref[...] = dinv
    g0_ref[...] = dinv * h0_ref[...]


_prep = pl.pallas_call(
    _prep_body,
    out_shape=(jax.ShapeDtypeStruct((N, 1), jnp.float32),
               jax.ShapeDtypeStruct((N, D), jnp.float32)))


def _update_body(aggpair_ref, gprev_ref, h0_ref, dinv_ref, h_ref, g_ref):
    ssum = aggpair_ref[0:N, :] + gprev_ref[...]
    h = (1.0 - ALPHA) * (dinv_ref[...] * ssum) + ALPHA * h0_ref[...]
    h_ref[...] = h
    g_ref[...] = dinv_ref[...] * h


_update = pl.pallas_call(
    _update_body,
    out_shape=(jax.ShapeDtypeStruct((N, D), jnp.float32),
               jax.ShapeDtypeStruct((N, D), jnp.float32)))


def kernel(x, edge_index, W1, b1, W2, b2):
    src = edge_index[0].astype(jnp.int32)
    dst = edge_index[1].astype(jnp.int32)
    pad = EPT - E // NW
    src3 = jnp.pad(src.reshape(NW, E // NW), ((0, 0), (0, pad)),
                   constant_values=0).reshape(NW, NB, B)
    dst3 = jnp.pad(dst.reshape(NW, E // NW), ((0, 0), (0, pad)),
                   constant_values=NPAD - 1).reshape(NW, NB, B)
    zrows = jnp.zeros((RPT, D), jnp.float32)
    ones_tbl = jnp.ones((N, D), jnp.float32)

    h0 = _mlp(x, W1, b1[None, :], W2, b2[None, :])

    def body(k, carry):
        h, g, dinv = carry

        agg = _scatter(g, src3, dst3, zrows)

        def first(_):
            dinv0, g0 = _prep(agg, h0)
            return (h0, g0, dinv0)

        def later(_):
            h2, g2 = _update(agg, g, h0, dinv)
            return (h2, g2, dinv)

        return lax.cond(k == 0, first, later, None)

    init = (h0, ones_tbl, jnp.zeros((N, 1), jnp.float32))
    h, _, _ = lax.fori_loop(0, K + 1, body, init)
    return h
